# baseline (speedup 1.0000x reference)
import functools

import jax
import jax.numpy as jnp
from jax import lax
from jax.experimental import pallas as pl
from jax.experimental.pallas import tpu as pltpu
from jax.experimental.pallas import tpu_sc as plsc

N_USERS = 100000
N_ITEMS = 1000
BATCH = 16384

NC = 2
NS = 16
L = 16
NW = NC * NS
BPW = BATCH // NW

_mesh = plsc.VectorSubcoreMesh(core_axis_name="c", subcore_axis_name="s")


@functools.partial(
    pl.kernel,
    out_type=jax.ShapeDtypeStruct((BATCH,), jnp.float32),
    mesh=_mesh,
    compiler_params=pltpu.CompilerParams(needs_layout_passes=False),
    scratch_types=[
        pltpu.VMEM((BPW,), jnp.int32),
        pltpu.VMEM((BPW,), jnp.int32),
        pltpu.VMEM((8 * BPW,), jnp.int32),
        pltpu.VMEM((BPW, 128), jnp.float32),
        pltpu.VMEM((BPW,), jnp.float32),
        pltpu.SemaphoreType.DMA,
        pltpu.SemaphoreType.DMA,
    ],
)
def _gather_kernel(propT_hbm, user_hbm, item_hbm, out_hbm,
                   user_v, item_v, il8_v, rows_v, out_v, sem, sem2):
    wid = lax.axis_index("s") * NC + lax.axis_index("c")
    base = wid * BPW

    pltpu.sync_copy(user_hbm.at[pl.ds(base, BPW)], user_v)
    pltpu.sync_copy(item_hbm.at[pl.ds(base, BPW)], item_v)

    iota = lax.broadcasted_iota(jnp.int32, (L,), 0)
    for g in range(BPW // L):
        itv = item_v[pl.ds(g * L, L)]
        plsc.store_scatter(il8_v, [(iota + g * L) * 8], itv)

    H = BPW // (2 * L)

    def make_issue(s):
        def issue(g, carry):
            gbase = pl.multiple_of(g * L, L)
            uv = user_v[pl.ds(gbase, L)]
            wv = (uv >> 7) * 128
            for lane in range(L):
                p = g * L + lane
                lstart = pl.multiple_of(p * 8, 8)
                w = pl.multiple_of(wv[lane], 128)
                pltpu.async_copy(
                    propT_hbm.at[il8_v.at[pl.ds(lstart, 1)], pl.ds(w, 128)],
                    rows_v.at[pl.ds(p, 1), :],
                    s,
                )
            return carry
        return issue

    lax.fori_loop(0, H, make_issue(sem), jnp.int32(0))
    lax.fori_loop(H, 2 * H, make_issue(sem2), jnp.int32(0))

    pltpu.make_async_copy(
        propT_hbm.at[pl.ds(0, BPW // 2), pl.ds(0, 128)],
        rows_v.at[pl.ds(0, BPW // 2), :],
        sem,
    ).wait()

    for g in range(H):
        jv = iota + g * L
        col = user_v[pl.ds(g * L, L)] & 127
        out_v[pl.ds(g * L, L)] = plsc.load_gather(rows_v, [jv, col])

    pltpu.make_async_copy(
        propT_hbm.at[pl.ds(0, BPW // 2), pl.ds(0, 128)],
        rows_v.at[pl.ds(BPW // 2, BPW // 2), :],
        sem2,
    ).wait()

    for g in range(H, 2 * H):
        jv = iota + g * L
        col = user_v[pl.ds(g * L, L)] & 127
        out_v[pl.ds(g * L, L)] = plsc.load_gather(rows_v, [jv, col])

    pltpu.sync_copy(out_v, out_hbm.at[pl.ds(base, BPW)])


def kernel(prop, user_idx, item_idx):
    return _gather_kernel(jnp.swapaxes(prop, 0, 1), user_idx, item_idx)


# speedup vs baseline: 1.0166x; 1.0166x over previous
import functools

import jax
import jax.numpy as jnp
from jax import lax
from jax.experimental import pallas as pl
from jax.experimental.pallas import tpu as pltpu
from jax.experimental.pallas import tpu_sc as plsc

N_USERS = 100000
N_ITEMS = 1000
BATCH = 16384

NC = 2
NS = 16
L = 16
NW = NC * NS
BPW = BATCH // NW

_mesh = plsc.VectorSubcoreMesh(core_axis_name="c", subcore_axis_name="s")


@functools.partial(
    pl.kernel,
    out_type=jax.ShapeDtypeStruct((BATCH,), jnp.float32),
    mesh=_mesh,
    compiler_params=pltpu.CompilerParams(needs_layout_passes=False),
    scratch_types=[
        pltpu.VMEM((BPW,), jnp.int32),
        pltpu.VMEM((BPW,), jnp.int32),
        pltpu.VMEM((8 * BPW,), jnp.int32),
        pltpu.VMEM((BPW, 128), jnp.float32),
        pltpu.VMEM((BPW,), jnp.float32),
        pltpu.SemaphoreType.DMA,
    ],
)
def _gather_kernel(propT_hbm, user_hbm, item_hbm, out_hbm,
                   user_v, item_v, il8_v, rows_v, out_v, sem):
    wid = lax.axis_index("s") * NC + lax.axis_index("c")
    base = wid * BPW

    pltpu.sync_copy(user_hbm.at[pl.ds(base, BPW)], user_v)
    pltpu.sync_copy(item_hbm.at[pl.ds(base, BPW)], item_v)

    iota = lax.broadcasted_iota(jnp.int32, (L,), 0)
    for g in range(BPW // L):
        itv = item_v[pl.ds(g * L, L)]
        plsc.store_scatter(il8_v, [(iota + g * L) * 8], itv)

    def issue(g, carry):
        gbase = pl.multiple_of(g * L, L)
        uv = user_v[pl.ds(gbase, L)]
        wv = (uv >> 7) * 128
        for lane in range(L):
            p = g * L + lane
            lstart = pl.multiple_of(p * 8, 8)
            w = pl.multiple_of(wv[lane], 128)
            pltpu.async_copy(
                propT_hbm.at[il8_v.at[pl.ds(lstart, 1)], pl.ds(w, 128)],
                rows_v.at[pl.ds(p, 1), :],
                sem,
            )
        return carry

    lax.fori_loop(0, BPW // L, issue, jnp.int32(0))

    pltpu.make_async_copy(
        propT_hbm.at[pl.ds(0, BPW), pl.ds(0, 128)],
        rows_v,
        sem,
    ).wait()

    for g in range(BPW // L):
        jv = iota + g * L
        col = user_v[pl.ds(g * L, L)] & 127
        out_v[pl.ds(g * L, L)] = plsc.load_gather(rows_v, [jv, col])

    pltpu.sync_copy(out_v, out_hbm.at[pl.ds(base, BPW)])


def kernel(prop, user_idx, item_idx):
    return _gather_kernel(jnp.swapaxes(prop, 0, 1), user_idx, item_idx)


# speedup vs baseline: 1.0308x; 1.0140x over previous
import functools

import jax
import jax.numpy as jnp
from jax import lax
from jax.experimental import pallas as pl
from jax.experimental.pallas import tpu as pltpu
from jax.experimental.pallas import tpu_sc as plsc

N_USERS = 100000
N_ITEMS = 1000
BATCH = 16384

NC = 2
NS = 16
L = 16
NW = NC * NS
BPW = BATCH // NW

_mesh = plsc.VectorSubcoreMesh(core_axis_name="c", subcore_axis_name="s")


@functools.partial(
    pl.kernel,
    out_type=jax.ShapeDtypeStruct((BATCH,), jnp.float32),
    mesh=_mesh,
    compiler_params=pltpu.CompilerParams(needs_layout_passes=False),
    scratch_types=[
        pltpu.VMEM((BPW,), jnp.int32),
        pltpu.VMEM((BPW,), jnp.int32),
        pltpu.VMEM((8 * BPW,), jnp.int32),
        pltpu.VMEM((BPW, 128), jnp.float32),
        pltpu.VMEM((BPW,), jnp.float32),
        pltpu.SemaphoreType.DMA,
    ],
)
def _gather_kernel(propT_hbm, user_hbm, item_hbm, out_hbm,
                   user_v, item_v, il8_v, rows_v, out_v, sem):
    wid = lax.axis_index("s") * NC + lax.axis_index("c")
    base = wid * BPW

    cin1 = pltpu.async_copy(user_hbm.at[pl.ds(base, BPW)], user_v, sem)
    cin2 = pltpu.async_copy(item_hbm.at[pl.ds(base, BPW)], item_v, sem)
    cin1.wait()
    cin2.wait()

    iota = lax.broadcasted_iota(jnp.int32, (L,), 0)
    for g in range(BPW // L):
        itv = item_v[pl.ds(g * L, L)]
        plsc.store_scatter(il8_v, [(iota + g * L) * 8], itv)

    def issue(g, carry):
        gbase = pl.multiple_of(g * L, L)
        uv = user_v[pl.ds(gbase, L)]
        wv = (uv >> 7) * 128
        for lane in range(L):
            p = g * L + lane
            lstart = pl.multiple_of(p * 8, 8)
            w = pl.multiple_of(wv[lane], 128)
            pltpu.async_copy(
                propT_hbm.at[il8_v.at[pl.ds(lstart, 1)], pl.ds(w, 128)],
                rows_v.at[pl.ds(p, 1), :],
                sem,
            )
        return carry

    lax.fori_loop(0, BPW // L, issue, jnp.int32(0))

    pltpu.make_async_copy(
        propT_hbm.at[pl.ds(0, BPW), pl.ds(0, 128)],
        rows_v,
        sem,
    ).wait()

    for g in range(BPW // L):
        jv = iota + g * L
        col = user_v[pl.ds(g * L, L)] & 127
        out_v[pl.ds(g * L, L)] = plsc.load_gather(rows_v, [jv, col])

    pltpu.sync_copy(out_v, out_hbm.at[pl.ds(base, BPW)])


def kernel(prop, user_idx, item_idx):
    return _gather_kernel(jnp.swapaxes(prop, 0, 1), user_idx, item_idx)
